# baseline (speedup 1.0000x reference)
import functools

import jax
import jax.numpy as jnp
from jax import lax
from jax.experimental import pallas as pl
from jax.experimental.pallas import tpu as pltpu
from jax.experimental.pallas import tpu_sc as plsc

DIM = 64
BATCH = 16384
LANES = 16
IDX_CHUNK = 128


def _make_kernel(num_cores, num_subcores):
    nw = num_cores * num_subcores
    b_per_w = BATCH // nw
    n_chunks = b_per_w // IDX_CHUNK
    mesh = plsc.VectorSubcoreMesh(core_axis_name="c", subcore_axis_name="s")

    @functools.partial(
        pl.kernel,
        mesh=mesh,
        compiler_params=pltpu.CompilerParams(use_tc_tiling_on_sc=False),
        out_type=jax.ShapeDtypeStruct((BATCH,), jnp.float32),
        scratch_types=[
            pltpu.VMEM((n_chunks, IDX_CHUNK), jnp.int32),
            pltpu.VMEM((n_chunks, IDX_CHUNK), jnp.int32),
            pltpu.VMEM((n_chunks, IDX_CHUNK, DIM), jnp.float32),
            pltpu.VMEM((n_chunks, IDX_CHUNK, DIM), jnp.float32),
            pltpu.VMEM((b_per_w,), jnp.float32),
            pltpu.SemaphoreType.DMA,
        ],
    )
    def two_tower(uid_hbm, iid_hbm, uemb_hbm, iemb_hbm, out_hbm,
                  uid_v, iid_v, urows, vrows, out_v, sem):
        wid = lax.axis_index("s") * num_cores + lax.axis_index("c")
        base = wid * b_per_w
        chunk0 = wid * n_chunks
        pltpu.sync_copy(uid_hbm.at[pl.ds(chunk0, n_chunks)], uid_v)
        pltpu.sync_copy(iid_hbm.at[pl.ds(chunk0, n_chunks)], iid_v)
        copies = []
        for j in range(n_chunks):
            copies.append(pltpu.async_copy(uemb_hbm.at[uid_v.at[j]], urows.at[j], sem))
            copies.append(pltpu.async_copy(iemb_hbm.at[iid_v.at[j]], vrows.at[j], sem))
        for c in copies:
            c.wait()

        lane = lax.iota(jnp.int32, LANES)

        def row_dot(j, b):
            acc = urows[j, b, pl.ds(0, LANES)] * vrows[j, b, pl.ds(0, LANES)]
            for c in range(1, DIM // LANES):
                acc = acc + (urows[j, b, pl.ds(c * LANES, LANES)]
                             * vrows[j, b, pl.ds(c * LANES, LANES)])
            folded = acc + lax.rev(acc, (0,))
            s = folded[0]
            for k in range(1, LANES // 2):
                s = s + folded[k]
            return s

        for j in range(n_chunks):
            def body(g, carry, j=j):
                res = jnp.zeros((LANES,), jnp.float32)
                for r in range(LANES):
                    s = row_dot(j, g * LANES + r)
                    res = jnp.where(lane == r, s, res)
                out_v[pl.ds((j * IDX_CHUNK // LANES + g) * LANES, LANES)] = res
                return carry
            lax.fori_loop(0, IDX_CHUNK // LANES, body, 0)

        pltpu.sync_copy(out_v, out_hbm.at[pl.ds(base, b_per_w)])

    return two_tower


@jax.jit
def kernel(user_ids, item_ids, user_emb, item_emb):
    info = plsc.get_sparse_core_info()
    k = _make_kernel(info.num_cores, info.num_subcores)
    uid = user_ids.reshape(BATCH // IDX_CHUNK, IDX_CHUNK)
    iid = item_ids.reshape(BATCH // IDX_CHUNK, IDX_CHUNK)
    return k(uid, iid, user_emb, item_emb)
